# Initial kernel scaffold; baseline (speedup 1.0000x reference)
import jax
import jax.numpy as jnp
from jax.experimental import pallas as pl


def kernel(features, weights, bias, in_idx, out_idx):
    raise NotImplementedError("write your pallas kernel here")


# speedup vs baseline: 2.0026x; 2.0026x over previous
import functools

import jax
import jax.numpy as jnp
from jax import lax
from jax.experimental import pallas as pl
from jax.experimental.pallas import tpu as pltpu
from jax.experimental.pallas import tpu_sc as plsc

_LANES = 16
_EDGE_CHUNK = 8192


def _sparse_linear_sc(features, weights, bias, in_idx, out_idx):
    batch, n_in = features.shape
    n_out = bias.shape[0]
    nnz = in_idx.shape[0]

    info = plsc.get_sparse_core_info()
    num_workers = info.num_cores * info.num_subcores
    assert batch % num_workers == 0
    rows_per_worker = batch // num_workers

    n_chunks = -(-nnz // _EDGE_CHUNK)
    groups_per_chunk = _EDGE_CHUNK // _LANES

    mesh = plsc.VectorSubcoreMesh(core_axis_name="c", subcore_axis_name="s")

    @functools.partial(
        pl.kernel,
        out_type=jax.ShapeDtypeStruct((batch, n_out), jnp.float32),
        mesh=mesh,
        compiler_params=pltpu.CompilerParams(needs_layout_passes=False),
        scratch_types=(
            [pltpu.VMEM((n_in,), jnp.float32)] * rows_per_worker
            + [pltpu.VMEM((n_out,), jnp.float32)] * rows_per_worker
            + [
                pltpu.VMEM((_EDGE_CHUNK,), jnp.int32),
                pltpu.VMEM((_EDGE_CHUNK,), jnp.int32),
                pltpu.VMEM((_EDGE_CHUNK,), jnp.float32),
                pltpu.VMEM((_LANES,), jnp.int32),
            ]
        ),
    )
    def run(feat_hbm, w_hbm, bias_hbm, in_hbm, oidx_hbm, y_hbm, *scratch):
        feat_v = scratch[:rows_per_worker]
        acc_v = scratch[rows_per_worker:2 * rows_per_worker]
        in_v, out_v, w_v, key_v = scratch[2 * rows_per_worker:]
        wid = lax.axis_index("s") * info.num_cores + lax.axis_index("c")
        row0 = wid * rows_per_worker

        for r in range(rows_per_worker):
            pltpu.sync_copy(feat_hbm.at[row0 + r], feat_v[r])
            pltpu.sync_copy(bias_hbm, acc_v[r])

        lanes = lax.iota(jnp.int32, _LANES)
        lanes_p1 = jnp.minimum(lanes + 1, _LANES - 1)
        is_last_lane = lanes == (_LANES - 1)
        not_last_lane = jnp.logical_not(is_last_lane)

        def do_chunk(c, _):
            off = c * _EDGE_CHUNK
            pltpu.sync_copy(in_hbm.at[pl.ds(off, _EDGE_CHUNK)], in_v)
            pltpu.sync_copy(oidx_hbm.at[pl.ds(off, _EDGE_CHUNK)], out_v)
            pltpu.sync_copy(w_hbm.at[pl.ds(off, _EDGE_CHUNK)], w_v)

            def do_group(g, carry):
                s = g * _LANES
                o16 = out_v[pl.ds(s, _LANES)]
                i16 = in_v[pl.ds(s, _LANES)]
                w16 = w_v[pl.ds(s, _LANES)]

                ks0, _unused = plsc.sort_key_val(o16, lanes)
                key_v[...] = ks0
                knext = plsc.load_gather(key_v, [lanes_p1])
                seg_last = jnp.logical_or(ks0 != knext, is_last_lane)
                seg_last_inner = jnp.logical_and(seg_last, not_last_lane)

                for r in range(rows_per_worker):
                    vals = plsc.load_gather(feat_v[r], [i16])
                    prod = vals * w16
                    ks, ps = plsc.sort_key_val(o16, prod)
                    csum = plsc.cumsum(ps)
                    plsc.addupdate_scatter(acc_v[r], [ks], csum,
                                           mask=seg_last)
                    plsc.addupdate_scatter(acc_v[r], [knext], -csum,
                                           mask=seg_last_inner)
                return carry

            lax.fori_loop(0, groups_per_chunk, do_group, 0)
            return _

        lax.fori_loop(0, n_chunks, do_chunk, 0)

        for r in range(rows_per_worker):
            pltpu.sync_copy(acc_v[r], y_hbm.at[row0 + r])

    return run(features, weights, bias, in_idx, out_idx)


def kernel(features, weights, bias, in_idx, out_idx):
    nnz = in_idx.shape[0]
    padded = -(-nnz // _EDGE_CHUNK) * _EDGE_CHUNK
    pad = padded - nnz
    if pad:
        in_idx = jnp.pad(in_idx, (0, pad))
        out_idx = jnp.pad(out_idx, (0, pad))
        weights = jnp.pad(weights, (0, pad))
    return _sparse_linear_sc(features, weights, bias, in_idx, out_idx)


# speedup vs baseline: 8.0090x; 3.9994x over previous
import functools

import jax
import jax.numpy as jnp
from jax import lax
from jax.experimental import pallas as pl
from jax.experimental.pallas import tpu as pltpu
from jax.experimental.pallas import tpu_sc as plsc

_LANES = 16
_EDGE_CHUNK = 8192


def _sparse_linear_sc(features, weights, bias, in_idx, out_idx):
    batch, n_in = features.shape
    n_out = bias.shape[0]
    nnz = in_idx.shape[0]

    info = plsc.get_sparse_core_info()
    num_workers = info.num_cores * info.num_subcores
    assert batch % num_workers == 0
    rows_per_worker = batch // num_workers

    n_chunks = -(-nnz // _EDGE_CHUNK)
    groups_per_chunk = _EDGE_CHUNK // _LANES

    mesh = plsc.VectorSubcoreMesh(core_axis_name="c", subcore_axis_name="s")

    @functools.partial(
        pl.kernel,
        out_type=jax.ShapeDtypeStruct((batch, n_out), jnp.float32),
        mesh=mesh,
        compiler_params=pltpu.CompilerParams(needs_layout_passes=False),
        scratch_types=(
            [pltpu.VMEM((n_in,), jnp.float32)] * rows_per_worker
            + [pltpu.VMEM((n_out,), jnp.float32)] * rows_per_worker
            + [
                pltpu.VMEM((_EDGE_CHUNK,), jnp.int32),
                pltpu.VMEM((_EDGE_CHUNK,), jnp.int32),
                pltpu.VMEM((_EDGE_CHUNK,), jnp.float32),
            ]
        ),
    )
    def run(feat_hbm, w_hbm, bias_hbm, in_hbm, oidx_hbm, y_hbm, *scratch):
        feat_v = scratch[:rows_per_worker]
        acc_v = scratch[rows_per_worker:2 * rows_per_worker]
        in_v, out_v, w_v = scratch[2 * rows_per_worker:]
        wid = lax.axis_index("s") * info.num_cores + lax.axis_index("c")
        row0 = wid * rows_per_worker

        for r in range(rows_per_worker):
            pltpu.sync_copy(feat_hbm.at[row0 + r], feat_v[r])
            pltpu.sync_copy(bias_hbm, acc_v[r])

        lanes = lax.iota(jnp.int32, _LANES)
        lanes_p1 = jnp.minimum(lanes + 1, _LANES - 1)
        is_last_lane = lanes == (_LANES - 1)
        not_last_lane = jnp.logical_not(is_last_lane)

        def do_chunk(c, _):
            off = c * _EDGE_CHUNK
            pltpu.sync_copy(in_hbm.at[pl.ds(off, _EDGE_CHUNK)], in_v)
            pltpu.sync_copy(oidx_hbm.at[pl.ds(off, _EDGE_CHUNK)], out_v)
            pltpu.sync_copy(w_hbm.at[pl.ds(off, _EDGE_CHUNK)], w_v)

            @plsc.parallel_loop(0, _EDGE_CHUNK, _LANES, unroll=4)
            def do_group(s):
                o16 = out_v[pl.ds(s, _LANES)]
                ks, perm = plsc.sort_key_val(o16, lanes)
                knext = ks.at[lanes_p1].get(mode="promise_in_bounds")
                seg_last = jnp.logical_or(ks != knext, is_last_lane)
                seg_last_inner = jnp.logical_and(seg_last, not_last_lane)
                perm_s = perm + s
                i16s = plsc.load_gather(in_v, [perm_s])
                w16s = plsc.load_gather(w_v, [perm_s])

                for r in range(rows_per_worker):
                    vals = plsc.load_gather(feat_v[r], [i16s])
                    csum = plsc.cumsum(vals * w16s)
                    plsc.addupdate_scatter(acc_v[r], [ks], csum,
                                           mask=seg_last)
                    plsc.addupdate_scatter(acc_v[r], [knext], -csum,
                                           mask=seg_last_inner)
            return _

        lax.fori_loop(0, n_chunks, do_chunk, 0)

        for r in range(rows_per_worker):
            pltpu.sync_copy(acc_v[r], y_hbm.at[row0 + r])

    return run(features, weights, bias, in_idx, out_idx)


def kernel(features, weights, bias, in_idx, out_idx):
    nnz = in_idx.shape[0]
    padded = -(-nnz // _EDGE_CHUNK) * _EDGE_CHUNK
    pad = padded - nnz
    if pad:
        in_idx = jnp.pad(in_idx, (0, pad))
        out_idx = jnp.pad(out_idx, (0, pad))
        weights = jnp.pad(weights, (0, pad))
    return _sparse_linear_sc(features, weights, bias, in_idx, out_idx)


# speedup vs baseline: 8.9260x; 1.1145x over previous
import functools

import jax
import jax.numpy as jnp
from jax import lax
from jax.experimental import pallas as pl
from jax.experimental.pallas import tpu as pltpu
from jax.experimental.pallas import tpu_sc as plsc

_LANES = 16
_EDGE_CHUNK = 8192


def _sparse_linear_sc(features, weights, bias, packed_idx):
    batch, n_in = features.shape
    n_out = bias.shape[0]
    nnz = packed_idx.shape[0]
    in_bits = (n_in - 1).bit_length()
    in_mask = (1 << in_bits) - 1

    info = plsc.get_sparse_core_info()
    num_workers = info.num_cores * info.num_subcores
    assert batch % num_workers == 0
    rows_per_worker = batch // num_workers

    n_chunks = -(-nnz // _EDGE_CHUNK)
    groups_per_chunk = _EDGE_CHUNK // _LANES

    mesh = plsc.VectorSubcoreMesh(core_axis_name="c", subcore_axis_name="s")

    @functools.partial(
        pl.kernel,
        out_type=jax.ShapeDtypeStruct((batch, n_out), jnp.float32),
        mesh=mesh,
        compiler_params=pltpu.CompilerParams(needs_layout_passes=False),
        scratch_types=(
            [pltpu.VMEM((n_in,), jnp.float32)] * rows_per_worker
            + [pltpu.VMEM((n_out,), jnp.float32)] * rows_per_worker
            + [
                pltpu.VMEM((_EDGE_CHUNK,), jnp.int32),
                pltpu.VMEM((_EDGE_CHUNK,), jnp.float32),
            ]
        ),
    )
    def run(feat_hbm, w_hbm, bias_hbm, pk_hbm, y_hbm, *scratch):
        feat_v = scratch[:rows_per_worker]
        acc_v = scratch[rows_per_worker:2 * rows_per_worker]
        pk_v, w_v = scratch[2 * rows_per_worker:]
        wid = lax.axis_index("s") * info.num_cores + lax.axis_index("c")
        row0 = wid * rows_per_worker

        for r in range(rows_per_worker):
            pltpu.sync_copy(feat_hbm.at[row0 + r], feat_v[r])
            pltpu.sync_copy(bias_hbm, acc_v[r])

        lanes = lax.iota(jnp.int32, _LANES)
        lanes_p1 = jnp.minimum(lanes + 1, _LANES - 1)
        is_last_lane = lanes == (_LANES - 1)
        not_last_lane = jnp.logical_not(is_last_lane)

        def do_chunk(c, _):
            off = c * _EDGE_CHUNK
            pltpu.sync_copy(pk_hbm.at[pl.ds(off, _EDGE_CHUNK)], pk_v)
            pltpu.sync_copy(w_hbm.at[pl.ds(off, _EDGE_CHUNK)], w_v)

            @plsc.parallel_loop(0, _EDGE_CHUNK, _LANES, unroll=8)
            def do_group(s):
                pk16 = pk_v[pl.ds(s, _LANES)]
                w16 = w_v[pl.ds(s, _LANES)]
                ks, ws = plsc.sort_key_val(pk16, w16)
                i16s = ks & in_mask
                o16s = ks >> in_bits
                onext = ks.at[lanes_p1].get(mode="promise_in_bounds") >> in_bits
                seg_last = jnp.logical_or(o16s != onext, is_last_lane)
                seg_last_inner = jnp.logical_and(seg_last, not_last_lane)

                for r in range(rows_per_worker):
                    vals = plsc.load_gather(feat_v[r], [i16s])
                    csum = plsc.cumsum(vals * ws)
                    plsc.addupdate_scatter(acc_v[r], [o16s], csum,
                                           mask=seg_last)
                    plsc.addupdate_scatter(acc_v[r], [onext], -csum,
                                           mask=seg_last_inner)
            return _

        lax.fori_loop(0, n_chunks, do_chunk, 0)

        for r in range(rows_per_worker):
            pltpu.sync_copy(acc_v[r], y_hbm.at[row0 + r])

    return run(features, weights, bias, packed_idx)


def kernel(features, weights, bias, in_idx, out_idx):
    n_in = features.shape[1]
    n_out = bias.shape[0]
    in_bits = (n_in - 1).bit_length()
    assert in_bits + (n_out - 1).bit_length() <= 31
    packed_idx = (out_idx << in_bits) | in_idx
    nnz = in_idx.shape[0]
    padded = -(-nnz // _EDGE_CHUNK) * _EDGE_CHUNK
    pad = padded - nnz
    if pad:
        packed_idx = jnp.pad(packed_idx, (0, pad))
        weights = jnp.pad(weights, (0, pad))
    return _sparse_linear_sc(features, weights, bias, packed_idx)
